# Initial kernel scaffold; baseline (speedup 1.0000x reference)
import jax
import jax.numpy as jnp
from jax.experimental import pallas as pl


def kernel(x, table, W, b):
    raise NotImplementedError("write your pallas kernel here")


# speedup vs baseline: 2.9963x; 2.9963x over previous
import functools

import jax
import jax.numpy as jnp
from jax import lax
from jax.experimental import pallas as pl
from jax.experimental.pallas import tpu as pltpu
from jax.experimental.pallas import tpu_sc as plsc

NC = 2
NS = 16
NW = NC * NS
LANES = 16
CHUNK = 128


@jax.jit
def _sc_embed_pool_linear(x_t, table, w2, b16):
  L, B = x_t.shape
  V, D = table.shape
  bpw = B // NW
  nchunk = bpw // CHUNK
  nblk = bpw // LANES

  mesh = plsc.VectorSubcoreMesh(core_axis_name="c", subcore_axis_name="s")

  @functools.partial(
      pl.kernel,
      out_type=jax.ShapeDtypeStruct((B,), jnp.float32),
      mesh=mesh,
      compiler_params=pltpu.CompilerParams(
          needs_layout_passes=False, use_tc_tiling_on_sc=False),
      scratch_types=[
          pltpu.VMEM((L, bpw), jnp.int32),
          pltpu.VMEM((bpw, D), jnp.float32),
          pltpu.VMEM((bpw, D), jnp.float32),
          pltpu.VMEM((2 * D,), jnp.float32),
          pltpu.VMEM((LANES,), jnp.float32),
          pltpu.VMEM((bpw,), jnp.float32),
          pltpu.SemaphoreType.DMA,
          pltpu.SemaphoreType.DMA,
      ],
  )
  def k(x_hbm, table_hbm, w_hbm, b_hbm, out_hbm, x_v, acc0, acc1, w_v, b_v,
        out_v, sem0, sem1):
    wid = lax.axis_index("s") * NC + lax.axis_index("c")
    base = wid * bpw
    pltpu.sync_copy(x_hbm.at[:, pl.ds(base, bpw)], x_v)
    pltpu.sync_copy(w_hbm, w_v)
    pltpu.sync_copy(b_hbm, b_v)

    def fire(l, acc, sem, add):
      for c in range(nchunk):
        idx = x_v.at[l, pl.ds(c * CHUNK, CHUNK)]
        dst = acc.at[pl.ds(c * CHUNK, CHUNK), :]
        pltpu.async_copy(table_hbm.at[idx], dst, sem, add=add)

    def drain(acc, sem):
      pltpu.make_async_copy(table_hbm.at[pl.ds(0, bpw), :], acc, sem).wait()

    fire(0, acc0, sem0, add=False)
    fire(1, acc1, sem1, add=False)

    def step(p, carry):
      l = 2 * p
      drain(acc0, sem0)
      fire(l, acc0, sem0, add=True)
      drain(acc1, sem1)
      fire(l + 1, acc1, sem1, add=True)
      return carry

    lax.fori_loop(1, L // 2, step, 0)
    if L % 2:
      drain(acc0, sem0)
      fire(L - 1, acc0, sem0, add=True)
    drain(acc0, sem0)
    drain(acc1, sem1)

    inv_l = jnp.float32(1.0 / L)
    bias_vec = b_v[pl.ds(0, LANES)]
    lanes = lax.iota(jnp.int32, LANES)

    def fin(blk, carry):
      c0 = blk * LANES
      row = c0 + lanes
      accv = jnp.zeros((LANES,), jnp.float32)
      for d0 in range(D):
        col = lax.rem(d0 + lanes, D)
        g = plsc.load_gather(acc0, [row, col]) + plsc.load_gather(
            acc1, [row, col])
        accv = accv + g * w_v[pl.ds(d0, LANES)]
      out_v[pl.ds(c0, LANES)] = accv * inv_l + bias_vec
      return carry

    lax.fori_loop(0, nblk, fin, 0)
    pltpu.sync_copy(out_v, out_hbm.at[pl.ds(base, bpw)])

  return k(x_t, table, w2, b16)


def kernel(x, table, W, b):
  x_t = jnp.transpose(x.astype(jnp.int32))
  w = W.reshape(-1).astype(jnp.float32)
  w2 = jnp.concatenate([w, w])
  b16 = jnp.broadcast_to(b.reshape(-1)[:1], (LANES,)).astype(jnp.float32)
  return _sc_embed_pool_linear(x_t, table, w2, b16)
